# baseline (speedup 1.0000x reference)
import functools

import jax
import jax.numpy as jnp
from jax import lax
from jax.experimental import pallas as pl
from jax.experimental.pallas import tpu as pltpu
from jax.experimental.pallas import tpu_sc as plsc

ORG_DIM = 80; C1 = 128; C2 = 64; K0 = 5; K1 = 3; K2 = 3
VQ_DIM = 64; VQ_NUM = 8192; CARD = 8; NG = 8
B = 16; T = 512
EPS = 1e-5

_HI = lax.Precision.HIGHEST

_SC_NC = 2
_SC_NS = 16
_SC_NW = _SC_NC * _SC_NS


def _conv_nch(x, w, groups=1):
    pad = w.shape[-1] // 2
    return lax.conv_general_dilated(x, w, (1,), [(pad, pad)],
                                    dimension_numbers=('NCH', 'OIH', 'NCH'),
                                    feature_group_count=groups)


def _gn_nch(x, g, b):
    n, c, l = x.shape
    xg = x.reshape(n, NG, c // NG, l)
    m = xg.mean(axis=(2, 3), keepdims=True)
    v = xg.var(axis=(2, 3), keepdims=True)
    xn = ((xg - m) / jnp.sqrt(v + EPS)).reshape(n, c, l)
    return xn * g[None, :, None] + b[None, :, None]


def _encode(x, p):
    relu = jax.nn.relu
    pool = lambda t: lax.reduce_window(t, -jnp.inf, lax.max,
                                       (1, 1, 2), (1, 1, 2), 'VALID')

    def resx(t):
        h = relu(_gn_nch(_conv_nch(t, p['rx_w1']), p['rx_g1'], p['rx_b1']))
        h = relu(_gn_nch(_conv_nch(h, p['rx_w2'], groups=CARD),
                         p['rx_g2'], p['rx_b2']))
        h = _gn_nch(_conv_nch(h, p['rx_w3']), p['rx_g3'], p['rx_b3'])
        return relu(t + h)

    out = relu(_gn_nch(_conv_nch(x, p['res0_w']), p['res0_g'], p['res0_b']))
    out = pool(resx(out))
    out = pool(resx(out))
    out = _conv_nch(out, p['res2_w'])
    out = ((out - p['res2_rm'][None, :, None])
           / jnp.sqrt(p['res2_rv'][None, :, None] + EPS)
           * p['res2_g'][None, :, None] + p['res2_b'][None, :, None])
    return out


def _mm(a, b):
    return lax.dot_general(a, b, (((1,), (0,)), ((), ())),
                           precision=_HI, preferred_element_type=jnp.float32)


def _mmc(a, b):
    return lax.dot_general(a.astype(jnp.bfloat16), b.astype(jnp.bfloat16),
                           (((1,), (0,)), ((), ())),
                           preferred_element_type=jnp.float32)


def _vq_body(h_ref, emb_ref, j_ref, loss_ref):
    b = pl.program_id(0)
    emb = emb_ref[...]
    w2s = jnp.sum(emb * emb, axis=1, keepdims=True)
    se = 0.0
    for i in range(2):
        h = h_ref[i]
        z2s = jnp.sum(h * h, axis=0, keepdims=True)
        dist = z2s + w2s - 2.0 * _mmc(emb, h)
        dmin = jnp.min(dist, axis=0, keepdims=True)
        ii = lax.broadcasted_iota(jnp.int32, dist.shape, 0)
        j_ref[i] = jnp.min(jnp.where(dist <= dmin, ii, VQ_NUM),
                           axis=0, keepdims=True)
        se = se + jnp.sum(dmin)

    @pl.when(b == 0)
    def _init():
        loss_ref[...] = jnp.zeros((1, 1), jnp.float32)
    acc = loss_ref[...] + se
    loss_ref[...] = jnp.where(b == B // 2 - 1, acc / (B * T // 4), acc)


def _make_gather():
    mesh = plsc.VectorSubcoreMesh(core_axis_name="c", subcore_axis_name="s",
                                  num_cores=_SC_NC, num_subcores=_SC_NS)
    n_per = (B * T // 4) // _SC_NW
    width = 2 * VQ_DIM

    @functools.partial(
        pl.kernel, mesh=mesh,
        out_type=jax.ShapeDtypeStruct((B * T // 4, width), jnp.float32),
        scratch_types=[
            pltpu.VMEM((n_per,), jnp.int32),
            pltpu.VMEM((n_per, width), jnp.float32),
            pltpu.SemaphoreType.DMA,
        ],
    )
    def gather_k(table_hbm, idx_hbm, out_hbm, idx_v, rows_v, sem):
        wid = lax.axis_index("s") * _SC_NC + lax.axis_index("c")
        base = wid * n_per
        pltpu.sync_copy(idx_hbm.at[pl.ds(base, n_per)], idx_v)
        pltpu.async_copy(table_hbm.at[idx_v], rows_v, sem).wait()
        pltpu.sync_copy(rows_v, out_hbm.at[pl.ds(base, n_per)])

    return gather_k


def _conv_taps(x, wtaps, k):
    c, t = x.shape
    pad = k // 2
    z = jnp.zeros((c, pad), jnp.float32)
    xp = jnp.concatenate([z, x, z], axis=1)
    acc = _mmc(wtaps[0], xp[:, 0:t])
    for i in range(1, k):
        acc = acc + _mmc(wtaps[i], xp[:, i:i + t])
    return acc


def _gn(x, a_grp, g, b):
    t = x.shape[1]
    s = jnp.concatenate([jnp.sum(x, axis=1, keepdims=True),
                         jnp.sum(x * x, axis=1, keepdims=True)], axis=1) / t
    ms = _mm(a_grp, s)
    mu = ms[:, 0:1]
    var = ms[:, 1:2] - mu * mu
    inv = 1.0 / jnp.sqrt(var + EPS)
    return (x - mu) * inv * g + b


def _bres(x, w1taps, g1, b1, w2taps, g2, b2, a128):
    h = jnp.maximum(_gn(_conv_taps(x, w1taps, K1), a128, g1, b1), 0.0)
    h = _gn(_conv_taps(h, w2taps, K1), a128, g2, b2)
    return jnp.maximum(x + h, 0.0)


def _dec_body(hq_ref, d2_ref, gd2_ref, bd2_ref,
              d11_ref, g11_ref, b11_ref, d12_ref, g12_ref, b12_ref,
              d0_ref, d0b_ref, a128_ref, r1_ref, r2_ref,
              out_ref):
    a128 = a128_ref[...]
    for i in range(4):
        hq = hq_ref[i]
        u = jnp.maximum(_gn(_conv_taps(hq, d2_ref[...], K2), a128,
                            gd2_ref[...], bd2_ref[...]), 0.0)
        u = _mm(u, r1_ref[...])
        br = lambda v: _bres(v, d11_ref[...], g11_ref[...], b11_ref[...],
                             d12_ref[...], g12_ref[...], b12_ref[...], a128)
        u = br(u)
        u = _mm(u, r2_ref[...])
        u = br(u)
        out_ref[i] = _conv_taps(u, d0_ref[...], K0) + d0b_ref[...]


def _prep_weights(p):
    f32 = jnp.float32
    col = lambda v: v.astype(f32).reshape(-1, 1)

    flipt = lambda w: jnp.transpose(jnp.flip(jnp.transpose(w, (1, 0, 2)),
                                             axis=-1), (2, 0, 1)).astype(jnp.bfloat16)
    d2 = flipt(p['d2_w'])
    d11 = flipt(p['d1_w1'])
    d12 = flipt(p['d1_w2'])
    d0 = flipt(p['d0_w'])

    def grp_avg(c):
        g = c // NG
        i = jnp.arange(c)
        return ((i[:, None] // g) == (i[None, :] // g)).astype(f32) / g
    a128 = grp_avg(C1)

    def rep2(t):
        r = jnp.zeros((t, 2 * t), f32)
        i = jnp.arange(t)
        r = r.at[i, 2 * i].set(1.0)
        return r.at[i, 2 * i + 1].set(1.0)
    r1 = rep2(T // 4)
    r2 = rep2(T // 2)

    return dict(
        a128=a128, r1=r1, r2=r2,
        d2=d2, gd2=col(p['d2_g']), bd2=col(p['d2_b']),
        d11=d11, g11=col(p['d1_g1']), b11=col(p['d1_b1']),
        d12=d12, g12=col(p['d1_g2']), b12=col(p['d1_b2']),
        d0=d0, d0b=col(p['d0_b']),
    )


def _full_spec(arr):
    nd = arr.ndim
    return pl.BlockSpec(arr.shape, lambda b: (0,) * nd)


def kernel(x, params):
    w = _prep_weights(params)
    emb = params['embed'].astype(jnp.float32)
    tq = T // 4

    h = _encode(x, params)

    j, loss = pl.pallas_call(
        _vq_body,
        grid=(B // 2,),
        in_specs=[pl.BlockSpec((2, C2, tq), lambda b: (b, 0, 0)),
                  _full_spec(emb)],
        out_specs=[pl.BlockSpec((2, 1, tq), lambda b: (b, 0, 0)),
                   pl.BlockSpec((1, 1), lambda b: (0, 0))],
        out_shape=[jax.ShapeDtypeStruct((B, 1, tq), jnp.int32),
                   jax.ShapeDtypeStruct((1, 1), jnp.float32)],
    )(h, emb)

    emb_pad = jnp.concatenate(
        [emb, jnp.zeros((VQ_NUM, VQ_DIM), jnp.float32)], axis=1)
    wj = _make_gather()(emb_pad, j.reshape(-1))[:, :VQ_DIM]
    hq = jnp.transpose(wj.reshape(B, tq, VQ_DIM), (0, 2, 1))

    dec_ins = [w['d2'], w['gd2'], w['bd2'], w['d11'], w['g11'], w['b11'],
               w['d12'], w['g12'], w['b12'], w['d0'], w['d0b'],
               w['a128'], w['r1'], w['r2']]
    out = pl.pallas_call(
        _dec_body,
        grid=(B // 4,),
        in_specs=[pl.BlockSpec((4, C2, tq), lambda b: (b, 0, 0))]
                 + [_full_spec(a) for a in dec_ins],
        out_specs=pl.BlockSpec((4, ORG_DIM, T), lambda b: (b, 0, 0)),
        out_shape=jax.ShapeDtypeStruct((B, ORG_DIM, T), jnp.float32),
    )(hq, *dec_ins)

    loss = loss[0, 0]
    return out, loss, loss


# speedup vs baseline: 1.0071x; 1.0071x over previous
import functools

import jax
import jax.numpy as jnp
from jax import lax
from jax.experimental import pallas as pl
from jax.experimental.pallas import tpu as pltpu
from jax.experimental.pallas import tpu_sc as plsc

ORG_DIM = 80; C1 = 128; C2 = 64; K0 = 5; K1 = 3; K2 = 3
VQ_DIM = 64; VQ_NUM = 8192; CARD = 8; NG = 8
B = 16; T = 512
EPS = 1e-5

_HI = lax.Precision.HIGHEST

_SC_NC = 2
_SC_NS = 16
_SC_NW = _SC_NC * _SC_NS


def _conv_nch(x, w, groups=1):
    pad = w.shape[-1] // 2
    return lax.conv_general_dilated(x, w, (1,), [(pad, pad)],
                                    dimension_numbers=('NCH', 'OIH', 'NCH'),
                                    feature_group_count=groups)


def _gn_nch(x, g, b):
    n, c, l = x.shape
    xg = x.reshape(n, NG, c // NG, l)
    m = xg.mean(axis=(2, 3), keepdims=True)
    v = xg.var(axis=(2, 3), keepdims=True)
    xn = ((xg - m) / jnp.sqrt(v + EPS)).reshape(n, c, l)
    return xn * g[None, :, None] + b[None, :, None]


def _encode(x, p):
    relu = jax.nn.relu
    pool = lambda t: lax.reduce_window(t, -jnp.inf, lax.max,
                                       (1, 1, 2), (1, 1, 2), 'VALID')

    def resx(t):
        h = relu(_gn_nch(_conv_nch(t, p['rx_w1']), p['rx_g1'], p['rx_b1']))
        h = relu(_gn_nch(_conv_nch(h, p['rx_w2'], groups=CARD),
                         p['rx_g2'], p['rx_b2']))
        h = _gn_nch(_conv_nch(h, p['rx_w3']), p['rx_g3'], p['rx_b3'])
        return relu(t + h)

    out = relu(_gn_nch(_conv_nch(x, p['res0_w']), p['res0_g'], p['res0_b']))
    out = pool(resx(out))
    out = pool(resx(out))
    out = _conv_nch(out, p['res2_w'])
    out = ((out - p['res2_rm'][None, :, None])
           / jnp.sqrt(p['res2_rv'][None, :, None] + EPS)
           * p['res2_g'][None, :, None] + p['res2_b'][None, :, None])
    return out


def _mm(a, b):
    return lax.dot_general(a, b, (((1,), (0,)), ((), ())),
                           precision=_HI, preferred_element_type=jnp.float32)


def _mmc(a, b):
    return lax.dot_general(a.astype(jnp.bfloat16), b.astype(jnp.bfloat16),
                           (((1,), (0,)), ((), ())),
                           preferred_element_type=jnp.float32)


def _vq_body(h_ref, emb_ref, j_ref, loss_ref):
    b = pl.program_id(0)
    emb = emb_ref[...]
    w2s = jnp.sum(emb * emb, axis=1, keepdims=True)
    se = 0.0
    for i in range(2):
        h = h_ref[i]
        z2s = jnp.sum(h * h, axis=0, keepdims=True)
        dist = z2s + w2s - 2.0 * _mmc(emb, h)
        dmin = jnp.min(dist, axis=0, keepdims=True)
        ii = lax.broadcasted_iota(jnp.int32, dist.shape, 0)
        j_ref[i] = jnp.min(jnp.where(dist <= dmin, ii, VQ_NUM),
                           axis=0, keepdims=True)
        se = se + jnp.sum(dmin)

    @pl.when(b == 0)
    def _init():
        loss_ref[...] = jnp.zeros((1, 1), jnp.float32)
    acc = loss_ref[...] + se
    loss_ref[...] = jnp.where(b == B // 2 - 1, acc / (B * T // 4), acc)


def _make_gather():
    mesh = plsc.VectorSubcoreMesh(core_axis_name="c", subcore_axis_name="s",
                                  num_cores=_SC_NC, num_subcores=_SC_NS)
    n_per = (B * T // 4) // _SC_NW
    width = 2 * VQ_DIM

    @functools.partial(
        pl.kernel, mesh=mesh,
        out_type=jax.ShapeDtypeStruct((B * T // 4, width), jnp.float32),
        scratch_types=[
            pltpu.VMEM((n_per,), jnp.int32),
            pltpu.VMEM((n_per, width), jnp.float32),
            pltpu.SemaphoreType.DMA,
        ],
    )
    def gather_k(table_hbm, idx_hbm, out_hbm, idx_v, rows_v, sem):
        wid = lax.axis_index("s") * _SC_NC + lax.axis_index("c")
        base = wid * n_per
        pltpu.sync_copy(idx_hbm.at[pl.ds(base, n_per)], idx_v)
        pltpu.async_copy(table_hbm.at[idx_v], rows_v, sem).wait()
        pltpu.sync_copy(rows_v, out_hbm.at[pl.ds(base, n_per)])

    return gather_k


def _conv_taps(x, wtaps, k):
    c, t = x.shape
    pad = k // 2
    z = jnp.zeros((c, pad), jnp.float32)
    xp = jnp.concatenate([z, x, z], axis=1)
    acc = _mmc(wtaps[0], xp[:, 0:t])
    for i in range(1, k):
        acc = acc + _mmc(wtaps[i], xp[:, i:i + t])
    return acc


def _gn(x, a_grp, g, b):
    t = x.shape[1]
    s = jnp.concatenate([jnp.sum(x, axis=1, keepdims=True),
                         jnp.sum(x * x, axis=1, keepdims=True)], axis=1) / t
    ms = _mm(a_grp, s)
    mu = ms[:, 0:1]
    var = ms[:, 1:2] - mu * mu
    inv = 1.0 / jnp.sqrt(var + EPS)
    return (x - mu) * inv * g + b


def _bres(x, w1taps, g1, b1, w2taps, g2, b2, a128):
    h = jnp.maximum(_gn(_conv_taps(x, w1taps, K1), a128, g1, b1), 0.0)
    h = _gn(_conv_taps(h, w2taps, K1), a128, g2, b2)
    return jnp.maximum(x + h, 0.0)


def _dec_body(hq_ref, d2_ref, gd2_ref, bd2_ref,
              d11_ref, g11_ref, b11_ref, d12_ref, g12_ref, b12_ref,
              d0_ref, d0b_ref, a128_ref, r1_ref, r2_ref,
              out_ref):
    a128 = a128_ref[...]
    for i in range(8):
        hq = hq_ref[i]
        u = jnp.maximum(_gn(_conv_taps(hq, d2_ref[...], K2), a128,
                            gd2_ref[...], bd2_ref[...]), 0.0)
        u = _mm(u, r1_ref[...])
        br = lambda v: _bres(v, d11_ref[...], g11_ref[...], b11_ref[...],
                             d12_ref[...], g12_ref[...], b12_ref[...], a128)
        u = br(u)
        u = _mm(u, r2_ref[...])
        u = br(u)
        out_ref[i] = _conv_taps(u, d0_ref[...], K0) + d0b_ref[...]


def _prep_weights(p):
    f32 = jnp.float32
    col = lambda v: v.astype(f32).reshape(-1, 1)

    flipt = lambda w: jnp.transpose(jnp.flip(jnp.transpose(w, (1, 0, 2)),
                                             axis=-1), (2, 0, 1)).astype(jnp.bfloat16)
    d2 = flipt(p['d2_w'])
    d11 = flipt(p['d1_w1'])
    d12 = flipt(p['d1_w2'])
    d0 = flipt(p['d0_w'])

    def grp_avg(c):
        g = c // NG
        i = jnp.arange(c)
        return ((i[:, None] // g) == (i[None, :] // g)).astype(f32) / g
    a128 = grp_avg(C1)

    def rep2(t):
        r = jnp.zeros((t, 2 * t), f32)
        i = jnp.arange(t)
        r = r.at[i, 2 * i].set(1.0)
        return r.at[i, 2 * i + 1].set(1.0)
    r1 = rep2(T // 4)
    r2 = rep2(T // 2)

    return dict(
        a128=a128, r1=r1, r2=r2,
        d2=d2, gd2=col(p['d2_g']), bd2=col(p['d2_b']),
        d11=d11, g11=col(p['d1_g1']), b11=col(p['d1_b1']),
        d12=d12, g12=col(p['d1_g2']), b12=col(p['d1_b2']),
        d0=d0, d0b=col(p['d0_b']),
    )


def _full_spec(arr):
    nd = arr.ndim
    return pl.BlockSpec(arr.shape, lambda b: (0,) * nd)


def kernel(x, params):
    w = _prep_weights(params)
    emb = params['embed'].astype(jnp.float32)
    tq = T // 4

    h = _encode(x, params)

    j, loss = pl.pallas_call(
        _vq_body,
        grid=(B // 2,),
        in_specs=[pl.BlockSpec((2, C2, tq), lambda b: (b, 0, 0)),
                  _full_spec(emb)],
        out_specs=[pl.BlockSpec((2, 1, tq), lambda b: (b, 0, 0)),
                   pl.BlockSpec((1, 1), lambda b: (0, 0))],
        out_shape=[jax.ShapeDtypeStruct((B, 1, tq), jnp.int32),
                   jax.ShapeDtypeStruct((1, 1), jnp.float32)],
    )(h, emb)

    emb_pad = jnp.concatenate(
        [emb, jnp.zeros((VQ_NUM, VQ_DIM), jnp.float32)], axis=1)
    wj = _make_gather()(emb_pad, j.reshape(-1))[:, :VQ_DIM]
    hq = jnp.transpose(wj.reshape(B, tq, VQ_DIM), (0, 2, 1))

    dec_ins = [w['d2'], w['gd2'], w['bd2'], w['d11'], w['g11'], w['b11'],
               w['d12'], w['g12'], w['b12'], w['d0'], w['d0b'],
               w['a128'], w['r1'], w['r2']]
    out = pl.pallas_call(
        _dec_body,
        grid=(B // 8,),
        in_specs=[pl.BlockSpec((8, C2, tq), lambda b: (b, 0, 0))]
                 + [_full_spec(a) for a in dec_ins],
        out_specs=pl.BlockSpec((8, ORG_DIM, T), lambda b: (b, 0, 0)),
        out_shape=jax.ShapeDtypeStruct((B, ORG_DIM, T), jnp.float32),
    )(hq, *dec_ins)

    loss = loss[0, 0]
    return out, loss, loss
